# Initial kernel scaffold; baseline (speedup 1.0000x reference)
import jax
import jax.numpy as jnp
from jax.experimental import pallas as pl


def kernel(pix_embedding, y, instance_label):
    raise NotImplementedError("write your pallas kernel here")


# speedup vs baseline: 270.4303x; 270.4303x over previous
import functools

import jax
import jax.numpy as jnp
from jax.experimental import pallas as pl

_DELTA_V = 0.5
_DELTA_D = 3.0
_PARAM_VAR = 1.0
_PARAM_DIST = 1.0
_PARAM_REG = 0.001
_D = 4


def _disc_loss_kernel(pix_ref, lab_ref, out_ref):
    pix = pix_ref[0]
    lab = lab_ref[0]

    masks = [(lab == c).astype(jnp.float32) for c in range(_D)]
    counts = [jnp.sum(m) for m in masks]
    seg = [[jnp.sum(pix[k] * masks[c]) for k in range(_D)] for c in range(_D)]

    cnt = [jnp.where(counts[c] > 0.0, counts[c], 1.0) for c in range(_D)]
    mu = [[jnp.where(c == 0, 0.0, seg[c][k] / cnt[c]) for k in range(_D)]
          for c in range(_D)]

    d2 = jnp.zeros_like(lab, dtype=jnp.float32)
    for k in range(_D):
        mu_exp_k = (masks[0] * mu[0][k] + masks[1] * mu[1][k]
                    + masks[2] * mu[2][k] + masks[3] * mu[3][k])
        diff = mu_exp_k - pix[k]
        d2 = d2 + diff * diff
    dist = jnp.sqrt(d2 + 1e-12)
    h = jnp.maximum(dist - _DELTA_V, 0.0)
    h2 = h * h
    s = [jnp.sum(h2 * masks[c]) for c in range(1, _D)]

    num_present = jnp.zeros((), jnp.float32)
    l_var_num = jnp.zeros((), jnp.float32)
    for idx in range(1, _D):
        w = (counts[idx] > 0.0).astype(jnp.float32)
        num_present = num_present + w
        l_var_num = l_var_num + w * (s[idx - 1] / cnt[idx])
    l_var = l_var_num / jnp.maximum(num_present, 1.0)

    sum_mask = jnp.zeros((), jnp.float32)
    sum_term = jnp.zeros((), jnp.float32)
    sum_inter = jnp.zeros((), jnp.float32)
    for a in range(_D):
        for b in range(_D):
            inter_abs = jnp.zeros((), jnp.float32)
            nrm2 = jnp.zeros((), jnp.float32)
            for k in range(_D):
                band_k = mu[b][k] * (mu[a][k] != 0.0).astype(jnp.float32)
                inter_k = mu[a][k] * (band_k != 0.0).astype(jnp.float32)
                diff_k = band_k - inter_k
                inter_abs = inter_abs + jnp.abs(diff_k)
                nrm2 = nrm2 + diff_k * diff_k
            maskp = (inter_abs != 0.0).astype(jnp.float32)
            nrm = jnp.sqrt(nrm2 + 1e-12)
            hp = jnp.maximum(2.0 * _DELTA_D - nrm, 0.0)
            sum_mask = sum_mask + maskp
            sum_term = sum_term + hp * hp * maskp
            sum_inter = sum_inter + inter_abs
    l_dist = jnp.where(sum_inter != 0.0,
                       sum_term / jnp.maximum(sum_mask, 1.0), 0.0)

    l_reg = jnp.zeros((), jnp.float32)
    for c in range(_D):
        n2 = jnp.zeros((), jnp.float32)
        for k in range(_D):
            n2 = n2 + mu[c][k] * mu[c][k]
        l_reg = l_reg + jnp.sqrt(n2 + 1e-12)
    l_reg = l_reg / _D

    total = _PARAM_VAR * l_var + _PARAM_DIST * l_dist + _PARAM_REG * l_reg
    loss = jnp.where(num_present > 0.0, total, 0.0)
    out_ref[0] = jnp.full((8, 128), loss, jnp.float32)


@functools.partial(jax.jit, static_argnames=("interpret",))
def _disc_loss(pix_embedding, instance_label, interpret=False):
    b = pix_embedding.shape[0]
    out = pl.pallas_call(
        _disc_loss_kernel,
        grid=(b,),
        in_specs=[
            pl.BlockSpec((1, _D, 512, 512), lambda i: (i, 0, 0, 0)),
            pl.BlockSpec((1, 512, 512), lambda i: (i, 0, 0)),
        ],
        out_specs=pl.BlockSpec((1, 8, 128), lambda i: (i, 0, 0)),
        out_shape=jax.ShapeDtypeStruct((b, 8, 128), jnp.float32),
        interpret=interpret,
    )(pix_embedding, instance_label)
    return jnp.mean(out[:, 0, 0])


def kernel(pix_embedding, y, instance_label):
    del y
    return _disc_loss(pix_embedding, instance_label)


# speedup vs baseline: 328.0595x; 1.2131x over previous
import functools

import jax
import jax.numpy as jnp
from jax.experimental import pallas as pl

_DELTA_V = 0.5
_DELTA_D = 3.0
_PARAM_VAR = 1.0
_PARAM_DIST = 1.0
_PARAM_REG = 0.001
_D = 4


def _disc_loss_kernel(pix_ref, lab_ref, out_ref):
    pix = pix_ref[0]
    lab = lab_ref[0]

    m = [lab == c for c in (1, 2, 3)]
    zf = jnp.zeros_like(pix[0])

    counts = [None] + [jnp.sum(jnp.where(mc, 1.0, 0.0)) for mc in m]
    seg = [[jnp.sum(jnp.where(m[c - 1], pix[k], zf)) for k in range(_D)]
           for c in range(1, _D)]

    cnt = [None] + [jnp.where(counts[c] > 0.0, counts[c], 1.0)
                    for c in range(1, _D)]
    mu = [[jnp.zeros((), jnp.float32)] * _D] + [
        [seg[c - 1][k] / cnt[c] for k in range(_D)] for c in range(1, _D)]

    d2 = zf
    for k in range(_D):
        mu_exp_k = jnp.where(m[1], mu[2][k], mu[1][k])
        mu_exp_k = jnp.where(m[2], mu[3][k], mu_exp_k)
        diff = mu_exp_k - pix[k]
        d2 = d2 + diff * diff
    dist = jnp.sqrt(d2 + 1e-12)
    h = jnp.maximum(dist - _DELTA_V, 0.0)
    h2 = h * h
    s = [jnp.sum(jnp.where(m[c - 1], h2, zf)) for c in range(1, _D)]

    num_present = jnp.zeros((), jnp.float32)
    l_var_num = jnp.zeros((), jnp.float32)
    for idx in range(1, _D):
        w = (counts[idx] > 0.0).astype(jnp.float32)
        num_present = num_present + w
        l_var_num = l_var_num + w * (s[idx - 1] / cnt[idx])
    l_var = l_var_num / jnp.maximum(num_present, 1.0)

    sum_mask = jnp.zeros((), jnp.float32)
    sum_term = jnp.zeros((), jnp.float32)
    sum_inter = jnp.zeros((), jnp.float32)
    for a in range(_D):
        for b in range(_D):
            inter_abs = jnp.zeros((), jnp.float32)
            nrm2 = jnp.zeros((), jnp.float32)
            for k in range(_D):
                band_k = mu[b][k] * (mu[a][k] != 0.0).astype(jnp.float32)
                inter_k = mu[a][k] * (band_k != 0.0).astype(jnp.float32)
                diff_k = band_k - inter_k
                inter_abs = inter_abs + jnp.abs(diff_k)
                nrm2 = nrm2 + diff_k * diff_k
            maskp = (inter_abs != 0.0).astype(jnp.float32)
            nrm = jnp.sqrt(nrm2 + 1e-12)
            hp = jnp.maximum(2.0 * _DELTA_D - nrm, 0.0)
            sum_mask = sum_mask + maskp
            sum_term = sum_term + hp * hp * maskp
            sum_inter = sum_inter + inter_abs
    l_dist = jnp.where(sum_inter != 0.0,
                       sum_term / jnp.maximum(sum_mask, 1.0), 0.0)

    l_reg = jnp.zeros((), jnp.float32)
    for c in range(_D):
        n2 = jnp.zeros((), jnp.float32)
        for k in range(_D):
            n2 = n2 + mu[c][k] * mu[c][k]
        l_reg = l_reg + jnp.sqrt(n2 + 1e-12)
    l_reg = l_reg / _D

    total = _PARAM_VAR * l_var + _PARAM_DIST * l_dist + _PARAM_REG * l_reg
    loss = jnp.where(num_present > 0.0, total, 0.0)
    out_ref[0] = jnp.full((8, 128), loss, jnp.float32)


@functools.partial(jax.jit, static_argnames=("interpret",))
def _disc_loss(pix_embedding, instance_label, interpret=False):
    b = pix_embedding.shape[0]
    out = pl.pallas_call(
        _disc_loss_kernel,
        grid=(b,),
        in_specs=[
            pl.BlockSpec((1, _D, 512, 512), lambda i: (i, 0, 0, 0)),
            pl.BlockSpec((1, 512, 512), lambda i: (i, 0, 0)),
        ],
        out_specs=pl.BlockSpec((1, 8, 128), lambda i: (i, 0, 0)),
        out_shape=jax.ShapeDtypeStruct((b, 8, 128), jnp.float32),
        interpret=interpret,
    )(pix_embedding, instance_label)
    return jnp.mean(out[:, 0, 0])


def kernel(pix_embedding, y, instance_label):
    del y
    return _disc_loss(pix_embedding, instance_label)
